# baseline (speedup 1.0000x reference)
import functools

import jax
import jax.numpy as jnp
from jax import lax
from jax.experimental import pallas as pl
from jax.experimental.pallas import tpu as pltpu
from jax.experimental.pallas import tpu_sc as plsc


def _swish(x):
    return x * jax.nn.sigmoid(x)


def _prep_tables(v, u, batchc, W1a, W1b, W1d, Wn1a, Wn1c, be1r, bn1r):
    n, d = v.shape
    g = u.shape[0]
    R = 1000
    grid = n // R

    def body(v_ref, u_ref, b_ref, w1a, w1b, w1d, wn1a, wn1c, be1_, bn1_,
             a2_ref, bt_ref, vn0_ref):
        oh = (b_ref[...] == lax.broadcasted_iota(jnp.int32, (1, g), 1)
              ).astype(jnp.float32)
        cu = jnp.dot(u_ref[...], w1d[...], preferred_element_type=jnp.float32)
        cn = jnp.dot(u_ref[...], wn1c[...], preferred_element_type=jnp.float32)
        vb = v_ref[...]
        def pack_u32(x):
            lo = jax.lax.bitcast_convert_type(
                x[:, :64].astype(jnp.bfloat16), jnp.uint16).astype(jnp.uint32)
            hi = jax.lax.bitcast_convert_type(
                x[:, 64:].astype(jnp.bfloat16), jnp.uint16).astype(jnp.uint32)
            return lo | (hi << jnp.uint32(16))

        a2_ref[...] = pack_u32(
            jnp.dot(vb, w1a[...], preferred_element_type=jnp.float32)
            + jnp.dot(oh, cu, preferred_element_type=jnp.float32)
            + be1_[...])
        bt_ref[...] = pack_u32(
            jnp.dot(vb, w1b[...], preferred_element_type=jnp.float32))
        vn0_ref[...] = (jnp.dot(vb, wn1a[...], preferred_element_type=jnp.float32)
                        + jnp.dot(oh, cn, preferred_element_type=jnp.float32)
                        + bn1_[...])

    wspec = pl.BlockSpec((128, 128), lambda i: (0, 0))
    bspec = pl.BlockSpec((1, 128), lambda i: (0, 0))
    return pl.pallas_call(
        body,
        grid=(grid,),
        in_specs=[
            pl.BlockSpec((R, d), lambda i: (i, 0)),
            pl.BlockSpec((g, d), lambda i: (0, 0)),
            pl.BlockSpec((R, 1), lambda i: (i, 0)),
            wspec, wspec, wspec, wspec, wspec, bspec, bspec,
        ],
        out_specs=[
            pl.BlockSpec((R, 64), lambda i: (i, 0)),
            pl.BlockSpec((R, 64), lambda i: (i, 0)),
            pl.BlockSpec((R, 128), lambda i: (i, 0)),
        ],
        out_shape=[
            jax.ShapeDtypeStruct((n, 64), jnp.uint32),
            jax.ShapeDtypeStruct((n, 64), jnp.uint32),
            jax.ShapeDtypeStruct((n, 128), jnp.float32),
        ],
    )(v, u, batchc, W1a, W1b, W1d, Wn1a, Wn1c, be1r, bn1r)


def _gather_pairsum(A2, Bt, rcp, nchpad, pc0, pc1):
    n, d = A2.shape
    assert 16 * (pc0 + pc1) == nchpad
    assert (pc0 - 4) % 3 == 0 and (pc1 - 4) % 3 == 0
    pmax = max(pc0, pc1)

    mesh = plsc.VectorSubcoreMesh(core_axis_name="core",
                                  subcore_axis_name="subcore")

    @functools.partial(
        pl.kernel,
        out_type=jax.ShapeDtypeStruct((nchpad * 128, d), jnp.uint32),
        mesh=mesh,
        compiler_params=pltpu.CompilerParams(needs_layout_passes=False,
                                             use_tc_tiling_on_sc=False),
        scratch_types=(
            [pltpu.VMEM((pmax, 2, 128), jnp.int32)]
            + [pltpu.VMEM((128, d), jnp.uint32)] * 6
            + [pltpu.SemaphoreType.DMA] * 6
        ),
    )
    def k(a_hbm, b_hbm, rc_hbm, o_hbm, idxall,
          a0, a1, a2, b0, b1, b2, sg0, sg1, sg2, sw0, sw1, sw2):
        cid = lax.axis_index("core")
        sid = lax.axis_index("subcore")
        pc = jnp.where(cid == 0, pc0, pc1)
        base = jnp.where(cid == 0, sid * pc0, 16 * pc0 + sid * pc1)
        avs, bvs = (a0, a1, a2), (b0, b1, b2)
        sgs, sws = (sg0, sg1, sg2), (sw0, sw1, sw2)

        pltpu.sync_copy(rc_hbm.at[pl.ds(base, pmax)], idxall)

        def g_descs(kk, s):
            return (pltpu.make_async_copy(a_hbm.at[idxall.at[kk, 0]],
                                          avs[s], sgs[s]),
                    pltpu.make_async_copy(b_hbm.at[idxall.at[kk, 1]],
                                          bvs[s], sgs[s]))

        def w_desc(kk, s):
            off = pl.multiple_of((base + kk) * 128, 128)
            return pltpu.make_async_copy(avs[s], o_hbm.at[pl.ds(off, 128)],
                                         sws[s])

        def add_slot(s):
            av, bv = avs[s], bvs[s]

            @pl.loop(0, 128)
            def _(r):
                for c in range(0, d, 16):
                    sl = (r, pl.ds(c, 16))
                    x = plsc.bitcast(av.at[sl][...], jnp.bfloat16)
                    y = plsc.bitcast(bv.at[sl][...], jnp.bfloat16)
                    av.at[sl][...] = plsc.bitcast(x + y, jnp.uint32)

        def issue_g(kk, s):
            for cp in g_descs(kk, s):
                cp.start()

        def wait_g(kk, s):
            for cp in g_descs(kk, s):
                cp.wait()

        issue_g(0, 0)
        wait_g(0, 0)
        issue_g(1, 1)
        add_slot(0)
        w_desc(0, 0).start()
        wait_g(1, 1)
        issue_g(2, 2)
        add_slot(1)
        w_desc(1, 1).start()

        @pl.loop(0, (pc - 4) // 3)
        def _(j):
            kk0 = 2 + 3 * j
            for b in range(3):
                kk = kk0 + b
                s = (2 + b) % 3
                wait_g(kk, s)
                w_desc(kk - 2, b).wait()
                issue_g(kk + 1, b)
                add_slot(s)
                w_desc(kk, s).start()

        k1, k2 = pc - 2, pc - 1
        wait_g(k1, 2)
        w_desc(k1 - 2, 0).wait()
        issue_g(k2, 0)
        add_slot(2)
        w_desc(k1, 2).start()
        wait_g(k2, 0)
        w_desc(k2 - 2, 1).wait()
        add_slot(0)
        w_desc(k2, 0).start()
        w_desc(k1, 2).wait()
        w_desc(k2, 0).wait()

    return k(A2, Bt, rcp)


def _edge_mlp(GAB, e, W1c, We2, be2r):
    m, d = e.shape
    RE = 4000
    grid = m // RE

    def body(gab_ref, e_ref, w1c, we2, be2_, o_ref):
        w = gab_ref[...]
        glo = jax.lax.bitcast_convert_type(w << jnp.uint32(16), jnp.float32)
        ghi = jax.lax.bitcast_convert_type(w & jnp.uint32(0xFFFF0000),
                                           jnp.float32)
        x = (jnp.concatenate([glo, ghi], axis=1)
             + jnp.dot(e_ref[...], w1c[...],
                       preferred_element_type=jnp.float32))
        h = _swish(x)
        y = jnp.dot(h, we2[...], preferred_element_type=jnp.float32) + be2_[...]
        o_ref[...] = _swish(y)

    return pl.pallas_call(
        body,
        grid=(grid,),
        in_specs=[
            pl.BlockSpec((RE, 64), lambda i: (i, 0)),
            pl.BlockSpec((RE, d), lambda i: (i, 0)),
            pl.BlockSpec((d, 128), lambda i: (0, 0)),
            pl.BlockSpec((128, 128), lambda i: (0, 0)),
            pl.BlockSpec((1, 128), lambda i: (0, 0)),
        ],
        out_specs=pl.BlockSpec((RE, 128), lambda i: (i, 0)),
        out_shape=jax.ShapeDtypeStruct((m, 128), jnp.float32),
    )(GAB, e, W1c, We2, be2r)


def _scatter_partials(ep, col3p, n_pad):
    m, d = ep.shape
    nreal = m // 128
    nchpad = col3p.shape[0]
    ntiles = 32
    per_tile = nchpad // ntiles
    assert per_tile * ntiles == nchpad and per_tile % 2 == 0
    rows_per_sub = n_pad // 16
    zfull = rows_per_sub // 128
    zrem = rows_per_sub - zfull * 128

    mesh = plsc.VectorSubcoreMesh(core_axis_name="core",
                                  subcore_axis_name="subcore")

    @functools.partial(
        pl.kernel,
        out_type=jax.ShapeDtypeStruct((2, n_pad, d), jnp.float32),
        mesh=mesh,
        scratch_types=(
            [pltpu.VMEM((per_tile, 1, 128), jnp.int32)]
            + [pltpu.VMEM((128, d), jnp.float32)] * 2
            + [pltpu.SemaphoreType.DMA] * 4
            + [pltpu.VMEM_SHARED((n_pad, d), jnp.float32)]
        ),
    )
    def k(ep_hbm, col_hbm, o_hbm, idxall, d0, d1,
          sr0, sr1, ss0, ss1, agg_sh):
        cid = lax.axis_index("core")
        sid = lax.axis_index("subcore")
        wid = sid * 2 + cid
        base = wid * per_tile
        dvs = (d0, d1)
        srs, sss = (sr0, sr1), (ss0, ss1)

        pltpu.sync_copy(col_hbm.at[pl.ds(base, per_tile)], idxall)

        @pl.loop(0, 128)
        def _(r):
            for c in range(0, d, 16):
                d0.at[pl.ds(r, 1), pl.ds(c, 16)][...] = jnp.zeros(
                    (1, 16), jnp.float32)

        @pl.loop(0, zfull)
        def _(j):
            pltpu.sync_copy(d0,
                            agg_sh.at[pl.ds(sid * rows_per_sub + j * 128, 128)])

        if zrem:
            pltpu.sync_copy(
                d0.at[pl.ds(0, zrem)],
                agg_sh.at[pl.ds(sid * rows_per_sub + zfull * 128, zrem)])

        plsc.subcore_barrier()

        def r_desc(kk, s):
            cc = jnp.minimum(base + kk, nreal - 1)
            off = pl.multiple_of(cc * 128, 128)
            return pltpu.make_async_copy(ep_hbm.at[pl.ds(off, 128)],
                                         dvs[s], srs[s])

        def s_desc(kk, s):
            return pltpu.make_async_copy(dvs[s],
                                         agg_sh.at[idxall.at[kk, 0]], sss[s])

        def issue_s(kk, s):
            pltpu.async_copy(dvs[s], agg_sh.at[idxall.at[kk, 0]], sss[s],
                             add=True)

        r_desc(0, 0).start()
        r_desc(0, 0).wait()
        r_desc(1, 1).start()
        issue_s(0, 0)

        @pl.loop(0, (per_tile - 2) // 2)
        def _(j):
            for b in range(2):
                kk = 1 + 2 * j + b
                s = (1 + b) % 2
                r_desc(kk, s).wait()
                s_desc(kk - 1, 1 - s).wait()
                r_desc(kk + 1, 1 - s).start()
                issue_s(kk, s)

        k2 = per_tile - 1
        r_desc(k2, k2 % 2).wait()
        s_desc(k2 - 1, 1 - k2 % 2).wait()
        issue_s(k2, k2 % 2)
        s_desc(k2, k2 % 2).wait()

        plsc.subcore_barrier()
        pltpu.sync_copy(agg_sh.at[pl.ds(sid * rows_per_sub, rows_per_sub)],
                        o_hbm.at[cid, pl.ds(sid * rows_per_sub, rows_per_sub)])

    return k(ep, col3p)


def _node_mlp(P0, P1, Vn0, batchr, Wn1b, Wn2, bn2r, g):
    n, d = Vn0.shape
    R = 1000
    grid = n // R

    def body(p0_ref, p1_ref, vn0_ref, b_ref, wn1b, wn2, bn2_,
             vp_ref, agge_ref, aggn_ref):
        i = pl.program_id(0)
        agg = p0_ref[...] + p1_ref[...]
        x = vn0_ref[...] + jnp.dot(agg, wn1b[...],
                                   preferred_element_type=jnp.float32)
        h = _swish(x)
        vp = _swish(jnp.dot(h, wn2[...], preferred_element_type=jnp.float32)
                    + bn2_[...])
        vp_ref[...] = vp
        oht = (lax.broadcasted_iota(jnp.int32, (g, 1), 0) == b_ref[0]
               ).astype(jnp.float32)
        de = jnp.dot(oht, agg, preferred_element_type=jnp.float32)
        dn = jnp.dot(oht, vp, preferred_element_type=jnp.float32)

        @pl.when(i == 0)
        def _():
            agge_ref[...] = de
            aggn_ref[...] = dn

        @pl.when(i != 0)
        def _():
            agge_ref[...] = agge_ref[...] + de
            aggn_ref[...] = aggn_ref[...] + dn

    return pl.pallas_call(
        body,
        grid=(grid,),
        in_specs=[
            pl.BlockSpec((R, d), lambda i: (i, 0)),
            pl.BlockSpec((R, d), lambda i: (i, 0)),
            pl.BlockSpec((R, 128), lambda i: (i, 0)),
            pl.BlockSpec((1, 1, R), lambda i: (i, 0, 0)),
            pl.BlockSpec((128, 128), lambda i: (0, 0)),
            pl.BlockSpec((128, 128), lambda i: (0, 0)),
            pl.BlockSpec((1, 128), lambda i: (0, 0)),
        ],
        out_specs=[
            pl.BlockSpec((R, 128), lambda i: (i, 0)),
            pl.BlockSpec((g, 128), lambda i: (0, 0)),
            pl.BlockSpec((g, 128), lambda i: (0, 0)),
        ],
        out_shape=[
            jax.ShapeDtypeStruct((n, 128), jnp.float32),
            jax.ShapeDtypeStruct((g, 128), jnp.float32),
            jax.ShapeDtypeStruct((g, 128), jnp.float32),
        ],
    )(P0, P1, Vn0, batchr, Wn1b, Wn2, bn2r)


def _global_mlp(u, aggE, aggN, Wg1a, Wg1b, Wg1c, bg1r, Wg2, bg2r):
    g, d = u.shape

    def body(u_ref, ae_ref, an_ref, wa, wb, wc, bg1_, wg2, bg2_, o_ref):
        x = (jnp.dot(u_ref[...], wa[...], preferred_element_type=jnp.float32)
             + jnp.dot(ae_ref[...], wb[...], preferred_element_type=jnp.float32)
             + jnp.dot(an_ref[...], wc[...], preferred_element_type=jnp.float32)
             + bg1_[...])
        h = _swish(x)
        o_ref[...] = _swish(jnp.dot(h, wg2[...],
                                    preferred_element_type=jnp.float32)
                            + bg2_[...])

    return pl.pallas_call(
        body,
        out_shape=jax.ShapeDtypeStruct((g, 128), jnp.float32),
    )(u, aggE, aggN, Wg1a, Wg1b, Wg1c, bg1r, Wg2, bg2r)


def kernel(v, e, u, edge_index, batch,
           We1, be1, We2, be2, Wn1, bn1, Wn2, bn2, Wg1, bg1, Wg2, bg2):
    n, d = v.shape
    m = e.shape[0]
    g = u.shape[0]

    row = edge_index[0]
    col = edge_index[1]
    nchunks = m // 128
    nchpad = ((nchunks + 31) // 32) * 32
    pad = nchpad - nchunks
    n_pad = ((n + 127) // 128) * 128
    pc0, pc1 = 52, 28
    rc = jnp.stack([row.reshape(nchunks, 128), col.reshape(nchunks, 128)],
                   axis=1)
    rcp = jnp.pad(rc, ((0, pad + max(pc0, pc1)), (0, 0), (0, 0)))
    col3p = jnp.pad(col.reshape(nchunks, 1, 128),
                    ((0, pad), (0, 0), (0, 0)),
                    constant_values=n)
    batchc = batch.reshape(n, 1)
    batchr = batch.reshape(n // 1000, 1, 1000)

    W1a, W1b, W1c, W1d = We1[0:d], We1[d:2 * d], We1[2 * d:3 * d], We1[3 * d:]
    Wn1a, Wn1b, Wn1c = Wn1[0:d], Wn1[d:d + 128], Wn1[d + 128:]
    Wg1a, Wg1b, Wg1c = Wg1[0:d], Wg1[d:d + 128], Wg1[d + 128:]

    be1r = be1.reshape(1, -1)
    be2r = be2.reshape(1, -1)
    bn1r = bn1.reshape(1, -1)
    bn2r = bn2.reshape(1, -1)
    bg1r = bg1.reshape(1, -1)
    bg2r = bg2.reshape(1, -1)

    A2, Bt, Vn0 = _prep_tables(v, u, batchc, W1a, W1b, W1d, Wn1a, Wn1c,
                               be1r, bn1r)
    GAB = _gather_pairsum(A2, Bt, rcp, nchpad, pc0, pc1)
    ep = _edge_mlp(GAB, e, W1c, We2, be2r)
    P = _scatter_partials(ep, col3p, n_pad)
    vp, aggE, aggN = _node_mlp(P[0], P[1], Vn0, batchr, Wn1b, Wn2, bn2r, g)
    up = _global_mlp(u, aggE, aggN, Wg1a, Wg1b, Wg1c, bg1r, Wg2, bg2r)
    return (vp, ep, up)


# speedup vs baseline: 1.0291x; 1.0291x over previous
import functools

import jax
import jax.numpy as jnp
from jax import lax
from jax.experimental import pallas as pl
from jax.experimental.pallas import tpu as pltpu
from jax.experimental.pallas import tpu_sc as plsc


def _swish(x):
    return x * jax.nn.sigmoid(x)


def _prep_tables(v, u, batchc, W1a, W1b, W1d, Wn1a, Wn1c, be1r, bn1r):
    n, d = v.shape
    g = u.shape[0]
    R = 1000
    grid = n // R

    def body(v_ref, u_ref, b_ref, w1a, w1b, w1d, wn1a, wn1c, be1_, bn1_,
             a2_ref, bt_ref, vn0_ref):
        oh = (b_ref[...] == lax.broadcasted_iota(jnp.int32, (1, g), 1)
              ).astype(jnp.float32)
        cu = jnp.dot(u_ref[...], w1d[...], preferred_element_type=jnp.float32)
        cn = jnp.dot(u_ref[...], wn1c[...], preferred_element_type=jnp.float32)
        vb = v_ref[...]
        def pack_u32(x):
            lo = jax.lax.bitcast_convert_type(
                x[:, :64].astype(jnp.bfloat16), jnp.uint16).astype(jnp.uint32)
            hi = jax.lax.bitcast_convert_type(
                x[:, 64:].astype(jnp.bfloat16), jnp.uint16).astype(jnp.uint32)
            return lo | (hi << jnp.uint32(16))

        a2_ref[...] = pack_u32(
            jnp.dot(vb, w1a[...], preferred_element_type=jnp.float32)
            + jnp.dot(oh, cu, preferred_element_type=jnp.float32)
            + be1_[...])
        bt_ref[...] = pack_u32(
            jnp.dot(vb, w1b[...], preferred_element_type=jnp.float32))
        vn0_ref[...] = (jnp.dot(vb, wn1a[...], preferred_element_type=jnp.float32)
                        + jnp.dot(oh, cn, preferred_element_type=jnp.float32)
                        + bn1_[...])

    wspec = pl.BlockSpec((128, 128), lambda i: (0, 0))
    bspec = pl.BlockSpec((1, 128), lambda i: (0, 0))
    return pl.pallas_call(
        body,
        grid=(grid,),
        in_specs=[
            pl.BlockSpec((R, d), lambda i: (i, 0)),
            pl.BlockSpec((g, d), lambda i: (0, 0)),
            pl.BlockSpec((R, 1), lambda i: (i, 0)),
            wspec, wspec, wspec, wspec, wspec, bspec, bspec,
        ],
        out_specs=[
            pl.BlockSpec((R, 64), lambda i: (i, 0)),
            pl.BlockSpec((R, 64), lambda i: (i, 0)),
            pl.BlockSpec((R, 128), lambda i: (i, 0)),
        ],
        out_shape=[
            jax.ShapeDtypeStruct((n, 64), jnp.uint32),
            jax.ShapeDtypeStruct((n, 64), jnp.uint32),
            jax.ShapeDtypeStruct((n, 128), jnp.float32),
        ],
    )(v, u, batchc, W1a, W1b, W1d, Wn1a, Wn1c, be1r, bn1r)


def _gather_pairsum(A2, Bt, rcp, nchpad, pc0, pc1):
    n, d = A2.shape
    assert 16 * (pc0 + pc1) == nchpad
    assert (pc0 - 4) % 3 == 0 and (pc1 - 4) % 3 == 0
    pmax = max(pc0, pc1)

    mesh = plsc.VectorSubcoreMesh(core_axis_name="core",
                                  subcore_axis_name="subcore")

    @functools.partial(
        pl.kernel,
        out_type=jax.ShapeDtypeStruct((nchpad * 128, d), jnp.uint32),
        mesh=mesh,
        compiler_params=pltpu.CompilerParams(needs_layout_passes=False,
                                             use_tc_tiling_on_sc=False),
        scratch_types=(
            [pltpu.VMEM((pmax, 2, 128), jnp.int32)]
            + [pltpu.VMEM((128, d), jnp.uint32)] * 6
            + [pltpu.SemaphoreType.DMA] * 6
        ),
    )
    def k(a_hbm, b_hbm, rc_hbm, o_hbm, idxall,
          a0, a1, a2, b0, b1, b2, sg0, sg1, sg2, sw0, sw1, sw2):
        cid = lax.axis_index("core")
        sid = lax.axis_index("subcore")
        pc = jnp.where(cid == 0, pc0, pc1)
        base = jnp.where(cid == 0, sid * pc0, 16 * pc0 + sid * pc1)
        avs, bvs = (a0, a1, a2), (b0, b1, b2)
        sgs, sws = (sg0, sg1, sg2), (sw0, sw1, sw2)

        pltpu.sync_copy(rc_hbm.at[pl.ds(base, pmax)], idxall)

        def g_descs(kk, s):
            return (pltpu.make_async_copy(a_hbm.at[idxall.at[kk, 0]],
                                          avs[s], sgs[s]),
                    pltpu.make_async_copy(b_hbm.at[idxall.at[kk, 1]],
                                          bvs[s], sgs[s]))

        def w_desc(kk, s):
            off = pl.multiple_of((base + kk) * 128, 128)
            return pltpu.make_async_copy(avs[s], o_hbm.at[pl.ds(off, 128)],
                                         sws[s])

        def add_slot(s):
            av, bv = avs[s], bvs[s]

            @pl.loop(0, 128)
            def _(r):
                for c in range(0, d, 16):
                    sl = (r, pl.ds(c, 16))
                    x = plsc.bitcast(av.at[sl][...], jnp.bfloat16)
                    y = plsc.bitcast(bv.at[sl][...], jnp.bfloat16)
                    av.at[sl][...] = plsc.bitcast(x + y, jnp.uint32)

        def issue_g(kk, s):
            for cp in g_descs(kk, s):
                cp.start()

        def wait_g(kk, s):
            for cp in g_descs(kk, s):
                cp.wait()

        issue_g(0, 0)
        wait_g(0, 0)
        issue_g(1, 1)
        add_slot(0)
        w_desc(0, 0).start()
        wait_g(1, 1)
        issue_g(2, 2)
        add_slot(1)
        w_desc(1, 1).start()

        @pl.loop(0, (pc - 4) // 3)
        def _(j):
            kk0 = 2 + 3 * j
            for b in range(3):
                kk = kk0 + b
                s = (2 + b) % 3
                wait_g(kk, s)
                w_desc(kk - 2, b).wait()
                issue_g(kk + 1, b)
                add_slot(s)
                w_desc(kk, s).start()

        k1, k2 = pc - 2, pc - 1
        wait_g(k1, 2)
        w_desc(k1 - 2, 0).wait()
        issue_g(k2, 0)
        add_slot(2)
        w_desc(k1, 2).start()
        wait_g(k2, 0)
        w_desc(k2 - 2, 1).wait()
        add_slot(0)
        w_desc(k2, 0).start()
        w_desc(k1, 2).wait()
        w_desc(k2, 0).wait()

    return k(A2, Bt, rcp)


def _edge_mlp(GAB, e, W1c, We2, be2r):
    m, d = e.shape
    RE = 4000
    grid = m // RE

    def body(gab_ref, e_ref, w1c, we2, be2_, o_ref):
        w = gab_ref[...]
        glo = jax.lax.bitcast_convert_type(w << jnp.uint32(16), jnp.float32)
        ghi = jax.lax.bitcast_convert_type(w & jnp.uint32(0xFFFF0000),
                                           jnp.float32)
        x = (jnp.concatenate([glo, ghi], axis=1)
             + jnp.dot(e_ref[...], w1c[...],
                       preferred_element_type=jnp.float32))
        h = _swish(x)
        y = jnp.dot(h, we2[...], preferred_element_type=jnp.float32) + be2_[...]
        o_ref[...] = _swish(y)

    return pl.pallas_call(
        body,
        grid=(grid,),
        in_specs=[
            pl.BlockSpec((RE, 64), lambda i: (i, 0)),
            pl.BlockSpec((RE, d), lambda i: (i, 0)),
            pl.BlockSpec((d, 128), lambda i: (0, 0)),
            pl.BlockSpec((128, 128), lambda i: (0, 0)),
            pl.BlockSpec((1, 128), lambda i: (0, 0)),
        ],
        out_specs=pl.BlockSpec((RE, 128), lambda i: (i, 0)),
        out_shape=jax.ShapeDtypeStruct((m, 128), jnp.float32),
    )(GAB, e, W1c, We2, be2r)


def _scatter_partials(ep, col3p, n_pad):
    m, d = ep.shape
    nreal = m // 128
    nchpad = col3p.shape[0]
    ntiles = 32
    per_tile = nchpad // ntiles
    assert per_tile * ntiles == nchpad and per_tile % 2 == 0
    rows_per_sub = n_pad // 16
    zfull = rows_per_sub // 128
    zrem = rows_per_sub - zfull * 128

    mesh = plsc.VectorSubcoreMesh(core_axis_name="core",
                                  subcore_axis_name="subcore")

    @functools.partial(
        pl.kernel,
        out_type=jax.ShapeDtypeStruct((2, n_pad, d), jnp.float32),
        mesh=mesh,
        scratch_types=(
            [pltpu.VMEM((per_tile, 1, 128), jnp.int32)]
            + [pltpu.VMEM((128, d), jnp.float32)] * 2
            + [pltpu.SemaphoreType.DMA] * 4
            + [pltpu.VMEM_SHARED((n_pad, d), jnp.float32)]
        ),
    )
    def k(ep_hbm, col_hbm, o_hbm, idxall, d0, d1,
          sr0, sr1, ss0, ss1, agg_sh):
        cid = lax.axis_index("core")
        sid = lax.axis_index("subcore")
        wid = sid * 2 + cid
        base = wid * per_tile
        dvs = (d0, d1)
        srs, sss = (sr0, sr1), (ss0, ss1)

        pltpu.sync_copy(col_hbm.at[pl.ds(base, per_tile)], idxall)

        @pl.loop(0, 128)
        def _(r):
            for c in range(0, d, 16):
                d0.at[pl.ds(r, 1), pl.ds(c, 16)][...] = jnp.zeros(
                    (1, 16), jnp.float32)

        @pl.loop(0, zfull)
        def _(j):
            pltpu.sync_copy(d0,
                            agg_sh.at[pl.ds(sid * rows_per_sub + j * 128, 128)])

        if zrem:
            pltpu.sync_copy(
                d0.at[pl.ds(0, zrem)],
                agg_sh.at[pl.ds(sid * rows_per_sub + zfull * 128, zrem)])

        plsc.subcore_barrier()

        def r_desc(kk, s):
            cc = jnp.minimum(base + kk, nreal - 1)
            off = pl.multiple_of(cc * 128, 128)
            return pltpu.make_async_copy(ep_hbm.at[pl.ds(off, 128)],
                                         dvs[s], srs[s])

        def s_desc(kk, s):
            return pltpu.make_async_copy(dvs[s],
                                         agg_sh.at[idxall.at[kk, 0]], sss[s])

        def issue_s(kk, s):
            pltpu.async_copy(dvs[s], agg_sh.at[idxall.at[kk, 0]], sss[s],
                             add=True)

        r_desc(0, 0).start()
        r_desc(0, 0).wait()
        r_desc(1, 1).start()
        issue_s(0, 0)

        @pl.loop(0, (per_tile - 2) // 2)
        def _(j):
            for b in range(2):
                kk = 1 + 2 * j + b
                s = (1 + b) % 2
                r_desc(kk, s).wait()
                s_desc(kk - 1, 1 - s).wait()
                r_desc(kk + 1, 1 - s).start()
                issue_s(kk, s)

        k2 = per_tile - 1
        r_desc(k2, k2 % 2).wait()
        s_desc(k2 - 1, 1 - k2 % 2).wait()
        issue_s(k2, k2 % 2)
        s_desc(k2, k2 % 2).wait()

        plsc.subcore_barrier()
        pltpu.sync_copy(agg_sh.at[pl.ds(sid * rows_per_sub, rows_per_sub)],
                        o_hbm.at[cid, pl.ds(sid * rows_per_sub, rows_per_sub)])

    return k(ep, col3p)


def _node_mlp(P0, P1, Vn0, batchr, Wn1b, Wn2, bn2r, g):
    n, d = Vn0.shape
    R = 1000
    grid = n // R

    def body(p0_ref, p1_ref, vn0_ref, b_ref, wn1b, wn2, bn2_,
             vp_ref, agge_ref, aggn_ref):
        i = pl.program_id(0)
        agg = p0_ref[...] + p1_ref[...]
        x = vn0_ref[...] + jnp.dot(agg, wn1b[...],
                                   preferred_element_type=jnp.float32)
        h = _swish(x)
        vp = _swish(jnp.dot(h, wn2[...], preferred_element_type=jnp.float32)
                    + bn2_[...])
        vp_ref[...] = vp
        oht = (lax.broadcasted_iota(jnp.int32, (g, 1), 0) == b_ref[0]
               ).astype(jnp.float32)
        de = jnp.dot(oht, agg, preferred_element_type=jnp.float32)
        dn = jnp.dot(oht, vp, preferred_element_type=jnp.float32)

        @pl.when(i == 0)
        def _():
            agge_ref[...] = de
            aggn_ref[...] = dn

        @pl.when(i != 0)
        def _():
            agge_ref[...] = agge_ref[...] + de
            aggn_ref[...] = aggn_ref[...] + dn

    return pl.pallas_call(
        body,
        grid=(grid,),
        in_specs=[
            pl.BlockSpec((R, d), lambda i: (i, 0)),
            pl.BlockSpec((R, d), lambda i: (i, 0)),
            pl.BlockSpec((R, 128), lambda i: (i, 0)),
            pl.BlockSpec((1, 1, R), lambda i: (i, 0, 0)),
            pl.BlockSpec((128, 128), lambda i: (0, 0)),
            pl.BlockSpec((128, 128), lambda i: (0, 0)),
            pl.BlockSpec((1, 128), lambda i: (0, 0)),
        ],
        out_specs=[
            pl.BlockSpec((R, 128), lambda i: (i, 0)),
            pl.BlockSpec((g, 128), lambda i: (0, 0)),
            pl.BlockSpec((g, 128), lambda i: (0, 0)),
        ],
        out_shape=[
            jax.ShapeDtypeStruct((n, 128), jnp.float32),
            jax.ShapeDtypeStruct((g, 128), jnp.float32),
            jax.ShapeDtypeStruct((g, 128), jnp.float32),
        ],
    )(P0, P1, Vn0, batchr, Wn1b, Wn2, bn2r)


def _global_mlp(u, aggE, aggN, Wg1a, Wg1b, Wg1c, bg1r, Wg2, bg2r):
    g, d = u.shape

    def body(u_ref, ae_ref, an_ref, wa, wb, wc, bg1_, wg2, bg2_, o_ref):
        x = (jnp.dot(u_ref[...], wa[...], preferred_element_type=jnp.float32)
             + jnp.dot(ae_ref[...], wb[...], preferred_element_type=jnp.float32)
             + jnp.dot(an_ref[...], wc[...], preferred_element_type=jnp.float32)
             + bg1_[...])
        h = _swish(x)
        o_ref[...] = _swish(jnp.dot(h, wg2[...],
                                    preferred_element_type=jnp.float32)
                            + bg2_[...])

    return pl.pallas_call(
        body,
        out_shape=jax.ShapeDtypeStruct((g, 128), jnp.float32),
    )(u, aggE, aggN, Wg1a, Wg1b, Wg1c, bg1r, Wg2, bg2r)


def kernel(v, e, u, edge_index, batch,
           We1, be1, We2, be2, Wn1, bn1, Wn2, bn2, Wg1, bg1, Wg2, bg2):
    n, d = v.shape
    m = e.shape[0]
    g = u.shape[0]

    row = edge_index[0]
    col = edge_index[1]
    nchunks = m // 128
    nchpad = ((nchunks + 31) // 32) * 32
    pad = nchpad - nchunks
    n_pad = ((n + 127) // 128) * 128
    pc0, pc1 = 58, 22
    rc = jnp.stack([row.reshape(nchunks, 128), col.reshape(nchunks, 128)],
                   axis=1)
    rcp = jnp.pad(rc, ((0, pad + max(pc0, pc1)), (0, 0), (0, 0)))
    col3p = jnp.pad(col.reshape(nchunks, 1, 128),
                    ((0, pad), (0, 0), (0, 0)),
                    constant_values=n)
    batchc = batch.reshape(n, 1)
    batchr = batch.reshape(n // 1000, 1, 1000)

    W1a, W1b, W1c, W1d = We1[0:d], We1[d:2 * d], We1[2 * d:3 * d], We1[3 * d:]
    Wn1a, Wn1b, Wn1c = Wn1[0:d], Wn1[d:d + 128], Wn1[d + 128:]
    Wg1a, Wg1b, Wg1c = Wg1[0:d], Wg1[d:d + 128], Wg1[d + 128:]

    be1r = be1.reshape(1, -1)
    be2r = be2.reshape(1, -1)
    bn1r = bn1.reshape(1, -1)
    bn2r = bn2.reshape(1, -1)
    bg1r = bg1.reshape(1, -1)
    bg2r = bg2.reshape(1, -1)

    A2, Bt, Vn0 = _prep_tables(v, u, batchc, W1a, W1b, W1d, Wn1a, Wn1c,
                               be1r, bn1r)
    GAB = _gather_pairsum(A2, Bt, rcp, nchpad, pc0, pc1)
    ep = _edge_mlp(GAB, e, W1c, We2, be2r)
    P = _scatter_partials(ep, col3p, n_pad)
    vp, aggE, aggN = _node_mlp(P[0], P[1], Vn0, batchr, Wn1b, Wn2, bn2r, g)
    up = _global_mlp(u, aggE, aggN, Wg1a, Wg1b, Wg1c, bg1r, Wg2, bg2r)
    return (vp, ep, up)


# speedup vs baseline: 1.0398x; 1.0104x over previous
import functools

import jax
import jax.numpy as jnp
from jax import lax
from jax.experimental import pallas as pl
from jax.experimental.pallas import tpu as pltpu
from jax.experimental.pallas import tpu_sc as plsc


def _swish(x):
    return x * jax.nn.sigmoid(x)


def _prep_tables(v, u, batchc, W1a, W1b, W1d, Wn1a, Wn1c, be1r, bn1r):
    n, d = v.shape
    g = u.shape[0]
    R = 1000
    grid = n // R

    def body(v_ref, u_ref, b_ref, w1a, w1b, w1d, wn1a, wn1c, be1_, bn1_,
             a2_ref, bt_ref, vn0_ref):
        oh = (b_ref[...] == lax.broadcasted_iota(jnp.int32, (1, g), 1)
              ).astype(jnp.float32)
        cu = jnp.dot(u_ref[...], w1d[...], preferred_element_type=jnp.float32)
        cn = jnp.dot(u_ref[...], wn1c[...], preferred_element_type=jnp.float32)
        vb = v_ref[...]
        def pack_u32(x):
            lo = jax.lax.bitcast_convert_type(
                x[:, :64].astype(jnp.bfloat16), jnp.uint16).astype(jnp.uint32)
            hi = jax.lax.bitcast_convert_type(
                x[:, 64:].astype(jnp.bfloat16), jnp.uint16).astype(jnp.uint32)
            return lo | (hi << jnp.uint32(16))

        a2_ref[...] = pack_u32(
            jnp.dot(vb, w1a[...], preferred_element_type=jnp.float32)
            + jnp.dot(oh, cu, preferred_element_type=jnp.float32)
            + be1_[...])
        bt_ref[...] = pack_u32(
            jnp.dot(vb, w1b[...], preferred_element_type=jnp.float32))
        vn0_ref[...] = (jnp.dot(vb, wn1a[...], preferred_element_type=jnp.float32)
                        + jnp.dot(oh, cn, preferred_element_type=jnp.float32)
                        + bn1_[...])

    wspec = pl.BlockSpec((128, 128), lambda i: (0, 0))
    bspec = pl.BlockSpec((1, 128), lambda i: (0, 0))
    return pl.pallas_call(
        body,
        grid=(grid,),
        in_specs=[
            pl.BlockSpec((R, d), lambda i: (i, 0)),
            pl.BlockSpec((g, d), lambda i: (0, 0)),
            pl.BlockSpec((R, 1), lambda i: (i, 0)),
            wspec, wspec, wspec, wspec, wspec, bspec, bspec,
        ],
        out_specs=[
            pl.BlockSpec((R, 64), lambda i: (i, 0)),
            pl.BlockSpec((R, 64), lambda i: (i, 0)),
            pl.BlockSpec((R, 128), lambda i: (i, 0)),
        ],
        out_shape=[
            jax.ShapeDtypeStruct((n, 64), jnp.uint32),
            jax.ShapeDtypeStruct((n, 64), jnp.uint32),
            jax.ShapeDtypeStruct((n, 128), jnp.float32),
        ],
    )(v, u, batchc, W1a, W1b, W1d, Wn1a, Wn1c, be1r, bn1r)


def _gather_pairsum(A2, Bt, rcp, nchpad, pc0, pc1):
    n, d = A2.shape
    assert 16 * (pc0 + pc1) == nchpad
    assert (pc0 - 4) % 3 == 0 and (pc1 - 4) % 3 == 0
    pmax = max(pc0, pc1)

    mesh = plsc.VectorSubcoreMesh(core_axis_name="core",
                                  subcore_axis_name="subcore")

    @functools.partial(
        pl.kernel,
        out_type=jax.ShapeDtypeStruct((nchpad * 128, d), jnp.uint32),
        mesh=mesh,
        compiler_params=pltpu.CompilerParams(needs_layout_passes=False,
                                             use_tc_tiling_on_sc=False),
        scratch_types=(
            [pltpu.VMEM((pmax, 2, 128), jnp.int32)]
            + [pltpu.VMEM((128, d), jnp.uint32)] * 6
            + [pltpu.SemaphoreType.DMA] * 6
        ),
    )
    def k(a_hbm, b_hbm, rc_hbm, o_hbm, idxall,
          a0, a1, a2, b0, b1, b2, sg0, sg1, sg2, sw0, sw1, sw2):
        cid = lax.axis_index("core")
        sid = lax.axis_index("subcore")
        pc = jnp.where(cid == 0, pc0, pc1)
        base = jnp.where(cid == 0, sid * pc0, 16 * pc0 + sid * pc1)
        avs, bvs = (a0, a1, a2), (b0, b1, b2)
        sgs, sws = (sg0, sg1, sg2), (sw0, sw1, sw2)

        pltpu.sync_copy(rc_hbm.at[pl.ds(base, pmax)], idxall)

        def g_descs(kk, s):
            return (pltpu.make_async_copy(a_hbm.at[idxall.at[kk, 0]],
                                          avs[s], sgs[s]),
                    pltpu.make_async_copy(b_hbm.at[idxall.at[kk, 1]],
                                          bvs[s], sgs[s]))

        def w_desc(kk, s):
            off = pl.multiple_of((base + kk) * 128, 128)
            return pltpu.make_async_copy(avs[s], o_hbm.at[pl.ds(off, 128)],
                                         sws[s])

        def add_slot(s):
            av, bv = avs[s], bvs[s]

            @pl.loop(0, 128)
            def _(r):
                for c in range(0, d, 16):
                    sl = (r, pl.ds(c, 16))
                    x = plsc.bitcast(av.at[sl][...], jnp.bfloat16)
                    y = plsc.bitcast(bv.at[sl][...], jnp.bfloat16)
                    av.at[sl][...] = plsc.bitcast(x + y, jnp.uint32)

        def issue_g(kk, s):
            for cp in g_descs(kk, s):
                cp.start()

        def wait_g(kk, s):
            for cp in g_descs(kk, s):
                cp.wait()

        issue_g(0, 0)
        wait_g(0, 0)
        issue_g(1, 1)
        add_slot(0)
        w_desc(0, 0).start()
        wait_g(1, 1)
        issue_g(2, 2)
        add_slot(1)
        w_desc(1, 1).start()

        @pl.loop(0, (pc - 4) // 3)
        def _(j):
            kk0 = 2 + 3 * j
            for b in range(3):
                kk = kk0 + b
                s = (2 + b) % 3
                wait_g(kk, s)
                w_desc(kk - 2, b).wait()
                issue_g(kk + 1, b)
                add_slot(s)
                w_desc(kk, s).start()

        k1, k2 = pc - 2, pc - 1
        wait_g(k1, 2)
        w_desc(k1 - 2, 0).wait()
        issue_g(k2, 0)
        add_slot(2)
        w_desc(k1, 2).start()
        wait_g(k2, 0)
        w_desc(k2 - 2, 1).wait()
        add_slot(0)
        w_desc(k2, 0).start()
        w_desc(k1, 2).wait()
        w_desc(k2, 0).wait()

    return k(A2, Bt, rcp)


def _edge_mlp(GAB, e, W1c, We2, be2r):
    m, d = e.shape
    RE = 4000
    grid = m // RE

    def body(gab_ref, e_ref, w1c, we2, be2_, o_ref):
        w = gab_ref[...]
        glo = jax.lax.bitcast_convert_type(w << jnp.uint32(16), jnp.float32)
        ghi = jax.lax.bitcast_convert_type(w & jnp.uint32(0xFFFF0000),
                                           jnp.float32)
        x = (jnp.concatenate([glo, ghi], axis=1)
             + jnp.dot(e_ref[...], w1c[...],
                       preferred_element_type=jnp.float32))
        h = _swish(x)
        y = jnp.dot(h, we2[...], preferred_element_type=jnp.float32) + be2_[...]
        o_ref[...] = _swish(y)

    return pl.pallas_call(
        body,
        grid=(grid,),
        in_specs=[
            pl.BlockSpec((RE, 64), lambda i: (i, 0)),
            pl.BlockSpec((RE, d), lambda i: (i, 0)),
            pl.BlockSpec((d, 128), lambda i: (0, 0)),
            pl.BlockSpec((128, 128), lambda i: (0, 0)),
            pl.BlockSpec((1, 128), lambda i: (0, 0)),
        ],
        out_specs=pl.BlockSpec((RE, 128), lambda i: (i, 0)),
        out_shape=jax.ShapeDtypeStruct((m, 128), jnp.float32),
    )(GAB, e, W1c, We2, be2r)


def _scatter_partials(ep, col3p, n_pad):
    m, d = ep.shape
    nreal = m // 128
    nchpad = col3p.shape[0]
    ntiles = 32
    per_tile = nchpad // ntiles
    assert per_tile * ntiles == nchpad and per_tile % 2 == 0
    rows_per_sub = n_pad // 16
    zfull = rows_per_sub // 128
    zrem = rows_per_sub - zfull * 128

    mesh = plsc.VectorSubcoreMesh(core_axis_name="core",
                                  subcore_axis_name="subcore")

    @functools.partial(
        pl.kernel,
        out_type=jax.ShapeDtypeStruct((2, n_pad, d), jnp.float32),
        mesh=mesh,
        scratch_types=(
            [pltpu.VMEM((per_tile, 1, 128), jnp.int32)]
            + [pltpu.VMEM((128, d), jnp.float32)] * 2
            + [pltpu.SemaphoreType.DMA] * 4
            + [pltpu.VMEM_SHARED((n_pad, d), jnp.float32)]
        ),
    )
    def k(ep_hbm, col_hbm, o_hbm, idxall, d0, d1,
          sr0, sr1, ss0, ss1, agg_sh):
        cid = lax.axis_index("core")
        sid = lax.axis_index("subcore")
        wid = sid * 2 + cid
        base = wid * per_tile
        dvs = (d0, d1)
        srs, sss = (sr0, sr1), (ss0, ss1)

        pltpu.sync_copy(col_hbm.at[pl.ds(base, per_tile)], idxall)

        @pl.loop(0, 128)
        def _(r):
            for c in range(0, d, 16):
                d0.at[pl.ds(r, 1), pl.ds(c, 16)][...] = jnp.zeros(
                    (1, 16), jnp.float32)

        @pl.loop(0, zfull)
        def _(j):
            pltpu.sync_copy(d0,
                            agg_sh.at[pl.ds(sid * rows_per_sub + j * 128, 128)])

        if zrem:
            pltpu.sync_copy(
                d0.at[pl.ds(0, zrem)],
                agg_sh.at[pl.ds(sid * rows_per_sub + zfull * 128, zrem)])

        plsc.subcore_barrier()

        def r_desc(kk, s):
            cc = jnp.minimum(base + kk, nreal - 1)
            off = pl.multiple_of(cc * 128, 128)
            return pltpu.make_async_copy(ep_hbm.at[pl.ds(off, 128)],
                                         dvs[s], srs[s])

        def s_desc(kk, s):
            return pltpu.make_async_copy(dvs[s],
                                         agg_sh.at[idxall.at[kk, 0]], sss[s])

        def issue_s(kk, s):
            pltpu.async_copy(dvs[s], agg_sh.at[idxall.at[kk, 0]], sss[s],
                             add=True)

        r_desc(0, 0).start()
        r_desc(0, 0).wait()
        r_desc(1, 1).start()
        issue_s(0, 0)

        @pl.loop(0, (per_tile - 2) // 2)
        def _(j):
            for b in range(2):
                kk = 1 + 2 * j + b
                s = (1 + b) % 2
                r_desc(kk, s).wait()
                s_desc(kk - 1, 1 - s).wait()
                r_desc(kk + 1, 1 - s).start()
                issue_s(kk, s)

        k2 = per_tile - 1
        r_desc(k2, k2 % 2).wait()
        s_desc(k2 - 1, 1 - k2 % 2).wait()
        issue_s(k2, k2 % 2)
        s_desc(k2, k2 % 2).wait()

        plsc.subcore_barrier()
        pltpu.sync_copy(agg_sh.at[pl.ds(sid * rows_per_sub, rows_per_sub)],
                        o_hbm.at[cid, pl.ds(sid * rows_per_sub, rows_per_sub)])

    return k(ep, col3p)


def _node_mlp(P0, P1, Vn0, batchr, Wn1b, Wn2, bn2r, g):
    n, d = Vn0.shape
    R = 1000
    grid = n // R

    def body(p0_ref, p1_ref, vn0_ref, b_ref, wn1b, wn2, bn2_,
             vp_ref, agge_ref, aggn_ref):
        i = pl.program_id(0)
        agg = p0_ref[...] + p1_ref[...]
        x = vn0_ref[...] + jnp.dot(agg, wn1b[...],
                                   preferred_element_type=jnp.float32)
        h = _swish(x)
        vp = _swish(jnp.dot(h, wn2[...], preferred_element_type=jnp.float32)
                    + bn2_[...])
        vp_ref[...] = vp
        oht = (lax.broadcasted_iota(jnp.int32, (g, 1), 0) == b_ref[0]
               ).astype(jnp.float32)
        de = jnp.dot(oht, agg, preferred_element_type=jnp.float32)
        dn = jnp.dot(oht, vp, preferred_element_type=jnp.float32)

        @pl.when(i == 0)
        def _():
            agge_ref[...] = de
            aggn_ref[...] = dn

        @pl.when(i != 0)
        def _():
            agge_ref[...] = agge_ref[...] + de
            aggn_ref[...] = aggn_ref[...] + dn

    return pl.pallas_call(
        body,
        grid=(grid,),
        in_specs=[
            pl.BlockSpec((R, d), lambda i: (i, 0)),
            pl.BlockSpec((R, d), lambda i: (i, 0)),
            pl.BlockSpec((R, 128), lambda i: (i, 0)),
            pl.BlockSpec((1, 1, R), lambda i: (i, 0, 0)),
            pl.BlockSpec((128, 128), lambda i: (0, 0)),
            pl.BlockSpec((128, 128), lambda i: (0, 0)),
            pl.BlockSpec((1, 128), lambda i: (0, 0)),
        ],
        out_specs=[
            pl.BlockSpec((R, 128), lambda i: (i, 0)),
            pl.BlockSpec((g, 128), lambda i: (0, 0)),
            pl.BlockSpec((g, 128), lambda i: (0, 0)),
        ],
        out_shape=[
            jax.ShapeDtypeStruct((n, 128), jnp.float32),
            jax.ShapeDtypeStruct((g, 128), jnp.float32),
            jax.ShapeDtypeStruct((g, 128), jnp.float32),
        ],
    )(P0, P1, Vn0, batchr, Wn1b, Wn2, bn2r)


def _global_mlp(u, aggE, aggN, Wg1a, Wg1b, Wg1c, bg1r, Wg2, bg2r):
    g, d = u.shape

    def body(u_ref, ae_ref, an_ref, wa, wb, wc, bg1_, wg2, bg2_, o_ref):
        x = (jnp.dot(u_ref[...], wa[...], preferred_element_type=jnp.float32)
             + jnp.dot(ae_ref[...], wb[...], preferred_element_type=jnp.float32)
             + jnp.dot(an_ref[...], wc[...], preferred_element_type=jnp.float32)
             + bg1_[...])
        h = _swish(x)
        o_ref[...] = _swish(jnp.dot(h, wg2[...],
                                    preferred_element_type=jnp.float32)
                            + bg2_[...])

    return pl.pallas_call(
        body,
        out_shape=jax.ShapeDtypeStruct((g, 128), jnp.float32),
    )(u, aggE, aggN, Wg1a, Wg1b, Wg1c, bg1r, Wg2, bg2r)


def kernel(v, e, u, edge_index, batch,
           We1, be1, We2, be2, Wn1, bn1, Wn2, bn2, Wg1, bg1, Wg2, bg2):
    n, d = v.shape
    m = e.shape[0]
    g = u.shape[0]

    row = edge_index[0]
    col = edge_index[1]
    nchunks = m // 128
    nchpad = ((nchunks + 31) // 32) * 32
    pad = nchpad - nchunks
    n_pad = ((n + 127) // 128) * 128
    pc0, pc1 = 61, 19
    rc = jnp.stack([row.reshape(nchunks, 128), col.reshape(nchunks, 128)],
                   axis=1)
    rcp = jnp.pad(rc, ((0, pad + max(pc0, pc1)), (0, 0), (0, 0)))
    col3p = jnp.pad(col.reshape(nchunks, 1, 128),
                    ((0, pad), (0, 0), (0, 0)),
                    constant_values=n)
    batchc = batch.reshape(n, 1)
    batchr = batch.reshape(n // 1000, 1, 1000)

    W1a, W1b, W1c, W1d = We1[0:d], We1[d:2 * d], We1[2 * d:3 * d], We1[3 * d:]
    Wn1a, Wn1b, Wn1c = Wn1[0:d], Wn1[d:d + 128], Wn1[d + 128:]
    Wg1a, Wg1b, Wg1c = Wg1[0:d], Wg1[d:d + 128], Wg1[d + 128:]

    be1r = be1.reshape(1, -1)
    be2r = be2.reshape(1, -1)
    bn1r = bn1.reshape(1, -1)
    bn2r = bn2.reshape(1, -1)
    bg1r = bg1.reshape(1, -1)
    bg2r = bg2.reshape(1, -1)

    A2, Bt, Vn0 = _prep_tables(v, u, batchc, W1a, W1b, W1d, Wn1a, Wn1c,
                               be1r, bn1r)
    GAB = _gather_pairsum(A2, Bt, rcp, nchpad, pc0, pc1)
    ep = _edge_mlp(GAB, e, W1c, We2, be2r)
    P = _scatter_partials(ep, col3p, n_pad)
    vp, aggE, aggN = _node_mlp(P[0], P[1], Vn0, batchr, Wn1b, Wn2, bn2r, g)
    up = _global_mlp(u, aggE, aggN, Wg1a, Wg1b, Wg1c, bg1r, Wg2, bg2r)
    return (vp, ep, up)
